# baseline (speedup 1.0000x reference)
import functools

import jax
import jax.numpy as jnp
from jax import lax
from jax.experimental import pallas as pl
from jax.experimental.pallas import tpu as pltpu
from jax.experimental.pallas import tpu_sc as plsc

NC = 2
NS = 16
NW = NC * NS
LANES = 16
EPS = 1e-5
NGROUP = 4


def _stage0_call(feature, W1):
    B, CIN, N = feature.shape
    COUT = W1.shape[0]
    NT = 512
    NB = N // NT

    def body(f_ref, w_ref, lt_ref, s_ref, q_ref):
        nb = pl.program_id(1)
        fblk = f_ref[0]
        lt = lax.dot_general(fblk, w_ref[...], (((0,), (1,)), ((), ())),
                             preferred_element_type=jnp.float32)
        lt_ref[...] = lt
        s = jnp.sum(lt, axis=0)[None, None, :]
        q = jnp.sum(lt * lt, axis=0)[None, None, :]

        @pl.when(nb == 0)
        def _():
            s_ref[...] = s
            q_ref[...] = q

        @pl.when(nb != 0)
        def _():
            s_ref[...] = s_ref[...] + s
            q_ref[...] = q_ref[...] + q

    return pl.pallas_call(
        body,
        grid=(B, NB),
        in_specs=[
            pl.BlockSpec((1, CIN, NT), lambda b, nb: (b, 0, nb)),
            pl.BlockSpec((COUT, CIN), lambda b, nb: (0, 0)),
        ],
        out_specs=[
            pl.BlockSpec((NT, COUT), lambda b, nb: (b * NB + nb, 0)),
            pl.BlockSpec((1, 1, COUT), lambda b, nb: (b, 0, 0)),
            pl.BlockSpec((1, 1, COUT), lambda b, nb: (b, 0, 0)),
        ],
        out_shape=[
            jax.ShapeDtypeStruct((B * N, COUT), jnp.float32),
            jax.ShapeDtypeStruct((B, 1, COUT), jnp.float32),
            jax.ShapeDtypeStruct((B, 1, COUT), jnp.float32),
        ],
        compiler_params=pltpu.CompilerParams(
            dimension_semantics=("arbitrary", "arbitrary")),
    )(feature, W1)


def _sc_pass1_call(localT, idx2d, B, N, K, COUT):
    P = B * N
    PW = P // NW
    CP = 16
    RC = CP * K
    NG = RC // 128
    NCH = PW // CP
    NV = COUT // LANES
    mesh = plsc.VectorSubcoreMesh(core_axis_name="c", subcore_axis_name="s",
                                  num_cores=NC, num_subcores=NS)

    NIW = PW * K // 128

    @functools.partial(
        pl.kernel,
        out_type=jax.ShapeDtypeStruct((NW * 8, 128), jnp.float32),
        mesh=mesh,
        scratch_types=[
            pltpu.VMEM((NIW, 128), jnp.int32),
            pltpu.VMEM((2, RC, COUT), jnp.float32),
            pltpu.VMEM((2, CP, COUT), jnp.float32),
            pltpu.VMEM((8, 128), jnp.float32),
            pltpu.VMEM((2 * COUT // LANES, LANES), jnp.float32),
            pltpu.SemaphoreType.DMA,
            pltpu.SemaphoreType.DMA,
        ],
    )
    def k(lt_hbm, idx_hbm, part_hbm, idx_v, rows_v, cen_v, st_v, acc_v,
          sem0, sem1):
        wid = lax.axis_index("s") * NC + lax.axis_index("c")
        base_pt = wid * PW
        pltpu.sync_copy(idx_hbm.at[pl.ds(wid * NIW, NIW)], idx_v)
        sems = (sem0, sem1)
        z = jnp.zeros((LANES,), jnp.float32)
        for v in range(2 * NV):
            acc_v[v, :] = z

        def issue(ch, s):
            pt0 = base_pt + ch * CP
            pltpu.async_copy(lt_hbm.at[pl.ds(pt0, CP)], cen_v.at[s], sems[s])
            for g in range(NG):
                pltpu.async_copy(lt_hbm.at[idx_v.at[ch * NG + g]],
                                 rows_v.at[s].at[pl.ds(g * 128, 128)], sems[s])

        def drain(s):
            pltpu.make_async_copy(lt_hbm.at[pl.ds(0, CP)], cen_v.at[s],
                                  sems[s]).wait()
            pltpu.make_async_copy(lt_hbm.at[pl.ds(0, RC)], rows_v.at[s],
                                  sems[s]).wait()

        def compute(ch, s):
            def p_body(p, c2):
                cvs = [cen_v[s, p, pl.ds(v * LANES, LANES)] for v in range(NV)]
                for j in range(K):
                    r = p * K + j
                    for v in range(NV):
                        dv = rows_v[s, r, pl.ds(v * LANES, LANES)] - cvs[v]
                        plsc.addupdate(acc_v.at[v], dv)
                        plsc.addupdate(acc_v.at[NV + v], dv * dv)
                return c2

            lax.fori_loop(0, CP, p_body, 0)

        issue(0, 0)

        def pair_body(g2, c):
            ch0 = g2 * 2
            issue(ch0 + 1, 1)
            drain(0)
            compute(ch0, 0)
            issue(ch0 + 2, 0)
            drain(1)
            compute(ch0 + 1, 1)
            return c

        lax.fori_loop(0, NCH // 2 - 1, pair_body, 0)
        issue(NCH - 1, 1)
        drain(0)
        compute(NCH - 2, 0)
        drain(1)
        compute(NCH - 1, 1)

        h = NV // 2
        carry = [acc_v[v, :] for v in range(2 * NV)]
        s1a = sum(carry[1:h], carry[0])
        s1b = sum(carry[h + 1:NV], carry[h])
        s2a = sum(carry[NV + 1:NV + h], carry[NV])
        s2b = sum(carry[NV + h + 1:], carry[NV + h])
        st_v[0, pl.ds(0, LANES)] = s1a
        st_v[0, pl.ds(LANES, LANES)] = s1b
        st_v[0, pl.ds(2 * LANES, LANES)] = s2a
        st_v[0, pl.ds(3 * LANES, LANES)] = s2b
        pltpu.sync_copy(st_v, part_hbm.at[pl.ds(wid * 8, 8)])

    return k(localT, idx2d)


def _scale_call(localT, aK, B, N, COUT):
    NT = 512
    NB = N // NT

    def body(lt_ref, a_ref, out_ref):
        out_ref[...] = lt_ref[...] * a_ref[0]

    return pl.pallas_call(
        body,
        grid=(B, NB),
        in_specs=[
            pl.BlockSpec((NT, COUT), lambda b, nb: (b * NB + nb, 0)),
            pl.BlockSpec((1, 1, COUT), lambda b, nb: (b, 0, 0)),
        ],
        out_specs=pl.BlockSpec((NT, COUT), lambda b, nb: (b * NB + nb, 0)),
        out_shape=jax.ShapeDtypeStruct((B * N, COUT), jnp.float32),
        compiler_params=pltpu.CompilerParams(
            dimension_semantics=("arbitrary", "arbitrary")),
    )(localT, aK)


def _sc_pass2_call(scaledT, idx2d, bK, B, N, K, COUT):
    P = B * N
    PW = P // NW
    CP = 16
    RC = CP * K
    NG = RC // 128
    NCH = PW // CP
    NV = COUT // LANES
    mesh = plsc.VectorSubcoreMesh(core_axis_name="c", subcore_axis_name="s",
                                  num_cores=NC, num_subcores=NS)

    NIW = PW * K // 128

    @functools.partial(
        pl.kernel,
        out_type=jax.ShapeDtypeStruct((P, COUT), jnp.float32),
        mesh=mesh,
        scratch_types=[
            pltpu.VMEM((NIW, 128), jnp.int32),
            pltpu.VMEM((2, RC, COUT), jnp.float32),
            pltpu.VMEM((2, CP, COUT), jnp.float32),
            pltpu.VMEM((2 * CP, COUT), jnp.float32),
            pltpu.VMEM((8, COUT), jnp.float32),
            pltpu.SemaphoreType.DMA,
            pltpu.SemaphoreType.DMA,
        ],
    )
    def k(lt_hbm, idx_hbm, b_hbm, out_hbm,
          idx_v, rows_v, cen_v, ob_v, b_v, sem0, sem1):
        wid = lax.axis_index("s") * NC + lax.axis_index("c")
        base_pt = wid * PW
        bat = base_pt // N
        pltpu.sync_copy(b_hbm.at[bat], b_v)
        pltpu.sync_copy(idx_hbm.at[pl.ds(wid * NIW, NIW)], idx_v)
        bvs = [b_v[0, pl.ds(v * LANES, LANES)] for v in range(NV)]
        z = jnp.zeros((LANES,), jnp.float32)
        sems = (sem0, sem1)

        def issue(ch, s):
            pt0 = base_pt + ch * CP
            pltpu.async_copy(lt_hbm.at[pl.ds(pt0, CP)], cen_v.at[s], sems[s])
            for g in range(NG):
                pltpu.async_copy(lt_hbm.at[idx_v.at[ch * NG + g]],
                                 rows_v.at[s].at[pl.ds(g * 128, 128)], sems[s])

        def drain(s):
            pltpu.make_async_copy(lt_hbm.at[pl.ds(0, CP)], cen_v.at[s],
                                  sems[s]).wait()
            pltpu.make_async_copy(lt_hbm.at[pl.ds(0, RC)], rows_v.at[s],
                                  sems[s]).wait()

        def compute(ch, s):
            def p_body(p, c2):
                evs = [bvs[v] - cen_v[s, p, pl.ds(v * LANES, LANES)]
                       for v in range(NV)]
                accs = [z] * NV
                for j in range(K):
                    r = p * K + j
                    for v in range(NV):
                        t = rows_v[s, r, pl.ds(v * LANES, LANES)] + evs[v]
                        accs[v] = accs[v] + jnp.maximum(t, 0.0)
                for v in range(NV):
                    ob_v[s * CP + p, pl.ds(v * LANES, LANES)] = accs[v]
                return c2

            lax.fori_loop(0, CP, p_body, 0)

        def flush(ch0):
            pltpu.sync_copy(ob_v, out_hbm.at[pl.ds(base_pt + ch0 * CP, 2 * CP)])

        issue(0, 0)

        def pair_body(g2, c):
            ch0 = g2 * 2
            issue(ch0 + 1, 1)
            drain(0)
            compute(ch0, 0)
            issue(ch0 + 2, 0)
            drain(1)
            compute(ch0 + 1, 1)
            flush(ch0)
            return c

        lax.fori_loop(0, NCH // 2 - 1, pair_body, 0)
        issue(NCH - 1, 1)
        drain(0)
        compute(NCH - 2, 0)
        drain(1)
        compute(NCH - 1, 1)
        flush(NCH - 2)

    return k(scaledT, idx2d, bK)


def _stage4_call(localT, out_dif, a_cen, b_cen, B, N, COUT):
    NT = 512
    NB = N // NT
    eye = jnp.eye(NT, dtype=jnp.float32)

    def body(lt_ref, od_ref, ac_ref, bc_ref, eye_ref, out_ref):
        cen = jnp.maximum(lt_ref[...] * ac_ref[0] + bc_ref[0], 0.0)
        cenT = lax.dot_general(cen, eye_ref[...], (((0,), (0,)), ((), ())),
                               preferred_element_type=jnp.float32)
        difT = lax.dot_general(od_ref[...], eye_ref[...], (((0,), (0,)), ((), ())),
                               preferred_element_type=jnp.float32)
        out_ref[0, 0:COUT, :] = cenT
        out_ref[0, COUT:2 * COUT, :] = difT

    return pl.pallas_call(
        body,
        grid=(B, NB),
        in_specs=[
            pl.BlockSpec((NT, COUT), lambda b, nb: (b * NB + nb, 0)),
            pl.BlockSpec((NT, COUT), lambda b, nb: (b * NB + nb, 0)),
            pl.BlockSpec((1, 1, COUT), lambda b, nb: (b, 0, 0)),
            pl.BlockSpec((1, 1, COUT), lambda b, nb: (b, 0, 0)),
            pl.BlockSpec((NT, NT), lambda b, nb: (0, 0)),
        ],
        out_specs=pl.BlockSpec((1, 2 * COUT, NT), lambda b, nb: (b, 0, nb)),
        out_shape=jax.ShapeDtypeStruct((B, 2 * COUT, N), jnp.float32),
        compiler_params=pltpu.CompilerParams(
            dimension_semantics=("arbitrary", "arbitrary")),
    )(localT, out_dif, a_cen, b_cen, eye)


def kernel(feature, knn_inds, W1, W2, gamma, beta):
    B, CIN, N = feature.shape
    COUT = W1.shape[0]
    K = knn_inds.shape[2]
    Cg = 2 * COUT // NGROUP

    localT, sum_c, sumsq_c = _stage0_call(feature, W1)

    flat_idx = (knn_inds + (jnp.arange(B, dtype=jnp.int32) * N)[:, None, None])
    idx2d = flat_idx.reshape(B * N * K // 128, 128)

    part = _sc_pass1_call(localT, idx2d, B, N, K, COUT)

    cnt_cen = Cg * N
    cnt_dif = Cg * N * K
    mean_cen = sum_c.reshape(B, 2, Cg).sum(axis=2) / cnt_cen
    var_cen = sumsq_c.reshape(B, 2, Cg).sum(axis=2) / cnt_cen - mean_cen**2
    pw = (part.reshape(NW, 8, 128)[:, 0, :4 * LANES]
          .reshape(B, NW // B, 4, LANES).sum(axis=(1, 3)))
    mean_dif = pw[:, 0:2] / cnt_dif
    var_dif = pw[:, 2:4] / cnt_dif - mean_dif**2
    mean = jnp.concatenate([mean_cen, mean_dif], axis=1)
    inv = 1.0 / jnp.sqrt(jnp.concatenate([var_cen, var_dif], axis=1) + EPS)
    g_of_c = jnp.arange(2 * COUT) // Cg
    a_all = gamma[None, :] * inv[:, g_of_c]
    b_all = beta[None, :] - mean[:, g_of_c] * a_all
    a_cen, a_dif = a_all[:, :COUT], a_all[:, COUT:]
    b_cen, b_dif = b_all[:, :COUT], b_all[:, COUT:]

    aK = (a_dif / K).reshape(B, 1, COUT)
    bK8 = jnp.broadcast_to((b_dif / K)[:, None, :], (B, 8, COUT))
    scaledT = _scale_call(localT, aK, B, N, COUT)
    out_dif = _sc_pass2_call(scaledT, idx2d, bK8, B, N, K, COUT)

    return _stage4_call(localT, out_dif, a_cen.reshape(B, 1, COUT),
                        b_cen.reshape(B, 1, COUT), B, N, COUT)


# speedup vs baseline: 2.3749x; 2.3749x over previous
import functools

import jax
import jax.numpy as jnp
from jax import lax
from jax.experimental import pallas as pl
from jax.experimental.pallas import tpu as pltpu
from jax.experimental.pallas import tpu_sc as plsc

NC = 2
NS = 16
NW = NC * NS
LANES = 16
EPS = 1e-5
NGROUP = 4


def _stage0_call(feature, W1):
    B, CIN, N = feature.shape
    COUT = W1.shape[0]
    NT = 512
    NB = N // NT

    def body(f_ref, w_ref, lt_ref, s_ref, q_ref):
        nb = pl.program_id(1)
        fblk = f_ref[0]
        lt = lax.dot_general(fblk, w_ref[...], (((0,), (1,)), ((), ())),
                             preferred_element_type=jnp.float32)
        lt_ref[...] = lt
        s = jnp.sum(lt, axis=0)[None, None, :]
        q = jnp.sum(lt * lt, axis=0)[None, None, :]

        @pl.when(nb == 0)
        def _():
            s_ref[...] = s
            q_ref[...] = q

        @pl.when(nb != 0)
        def _():
            s_ref[...] = s_ref[...] + s
            q_ref[...] = q_ref[...] + q

    return pl.pallas_call(
        body,
        grid=(B, NB),
        in_specs=[
            pl.BlockSpec((1, CIN, NT), lambda b, nb: (b, 0, nb)),
            pl.BlockSpec((COUT, CIN), lambda b, nb: (0, 0)),
        ],
        out_specs=[
            pl.BlockSpec((NT, COUT), lambda b, nb: (b * NB + nb, 0)),
            pl.BlockSpec((1, 1, COUT), lambda b, nb: (b, 0, 0)),
            pl.BlockSpec((1, 1, COUT), lambda b, nb: (b, 0, 0)),
        ],
        out_shape=[
            jax.ShapeDtypeStruct((B * N, COUT), jnp.float32),
            jax.ShapeDtypeStruct((B, 1, COUT), jnp.float32),
            jax.ShapeDtypeStruct((B, 1, COUT), jnp.float32),
        ],
        compiler_params=pltpu.CompilerParams(
            dimension_semantics=("arbitrary", "arbitrary")),
    )(feature, W1)


def _sc_pass1_call(localT, idx2d, B, N, K, COUT):
    P = B * N
    PW = P // NW
    CP = 16
    RC = CP * K
    NG = RC // 128
    NCH = PW // CP
    NV = COUT // LANES
    mesh = plsc.VectorSubcoreMesh(core_axis_name="c", subcore_axis_name="s",
                                  num_cores=NC, num_subcores=NS)

    NIW = PW * K // 128

    @functools.partial(
        pl.kernel,
        out_type=jax.ShapeDtypeStruct((NW * 8, 128), jnp.float32),
        mesh=mesh,
        scratch_types=[
            pltpu.VMEM((NIW, 128), jnp.int32),
            pltpu.VMEM((2, RC, COUT), jnp.float32),
            pltpu.VMEM((2, CP, COUT), jnp.float32),
            pltpu.VMEM((8, 128), jnp.float32),
            pltpu.VMEM((2 * COUT // LANES, LANES), jnp.float32),
            pltpu.SemaphoreType.DMA,
            pltpu.SemaphoreType.DMA,
        ],
    )
    def k(lt_hbm, idx_hbm, part_hbm, idx_v, rows_v, cen_v, st_v, acc_v,
          sem0, sem1):
        wid = lax.axis_index("s") * NC + lax.axis_index("c")
        base_pt = wid * PW
        pltpu.sync_copy(idx_hbm.at[pl.ds(wid * NIW, NIW)], idx_v)
        sems = (sem0, sem1)
        z = jnp.zeros((LANES,), jnp.float32)
        for v in range(2 * NV):
            acc_v[v, :] = z

        def issue(ch, s):
            pt0 = base_pt + ch * CP
            pltpu.async_copy(lt_hbm.at[pl.ds(pt0, CP)], cen_v.at[s], sems[s])
            for g in range(NG):
                pltpu.async_copy(lt_hbm.at[idx_v.at[ch * NG + g]],
                                 rows_v.at[s].at[pl.ds(g * 128, 128)], sems[s])

        def drain(s):
            pltpu.make_async_copy(lt_hbm.at[pl.ds(0, CP)], cen_v.at[s],
                                  sems[s]).wait()
            pltpu.make_async_copy(lt_hbm.at[pl.ds(0, RC)], rows_v.at[s],
                                  sems[s]).wait()

        def compute(ch, s):
            def p_body(p, c2):
                cvs = [cen_v[s, p, pl.ds(v * LANES, LANES)] for v in range(NV)]
                s1 = [None] * NV
                s2 = [None] * NV
                for j in range(K):
                    r = p * K + j
                    for v in range(NV):
                        dv = rows_v[s, r, pl.ds(v * LANES, LANES)] - cvs[v]
                        sq = dv * dv
                        s1[v] = dv if j == 0 else s1[v] + dv
                        s2[v] = sq if j == 0 else s2[v] + sq
                for v in range(NV):
                    plsc.addupdate(acc_v.at[v], s1[v])
                    plsc.addupdate(acc_v.at[NV + v], s2[v])
                return c2

            lax.fori_loop(0, CP, p_body, 0)

        issue(0, 0)

        def pair_body(g2, c):
            ch0 = g2 * 2
            issue(ch0 + 1, 1)
            drain(0)
            compute(ch0, 0)
            issue(ch0 + 2, 0)
            drain(1)
            compute(ch0 + 1, 1)
            return c

        lax.fori_loop(0, NCH // 2 - 1, pair_body, 0)
        issue(NCH - 1, 1)
        drain(0)
        compute(NCH - 2, 0)
        drain(1)
        compute(NCH - 1, 1)

        h = NV // 2
        carry = [acc_v[v, :] for v in range(2 * NV)]
        s1a = sum(carry[1:h], carry[0])
        s1b = sum(carry[h + 1:NV], carry[h])
        s2a = sum(carry[NV + 1:NV + h], carry[NV])
        s2b = sum(carry[NV + h + 1:], carry[NV + h])
        st_v[0, pl.ds(0, LANES)] = s1a
        st_v[0, pl.ds(LANES, LANES)] = s1b
        st_v[0, pl.ds(2 * LANES, LANES)] = s2a
        st_v[0, pl.ds(3 * LANES, LANES)] = s2b
        pltpu.sync_copy(st_v, part_hbm.at[pl.ds(wid * 8, 8)])

    return k(localT, idx2d)


def _sc_pass2_call(localT, idx2d, a8, b8, B, N, K, COUT):
    P = B * N
    PW = P // NW
    CP = 16
    RC = CP * K
    NG = RC // 128
    NCH = PW // CP
    NV = COUT // LANES
    mesh = plsc.VectorSubcoreMesh(core_axis_name="c", subcore_axis_name="s",
                                  num_cores=NC, num_subcores=NS)

    NIW = PW * K // 128

    @functools.partial(
        pl.kernel,
        out_type=jax.ShapeDtypeStruct((P, COUT), jnp.float32),
        mesh=mesh,
        scratch_types=[
            pltpu.VMEM((NIW, 128), jnp.int32),
            pltpu.VMEM((2, RC, COUT), jnp.float32),
            pltpu.VMEM((2, CP, COUT), jnp.float32),
            pltpu.VMEM((2 * CP, COUT), jnp.float32),
            pltpu.VMEM((8, COUT), jnp.float32),
            pltpu.VMEM((8, COUT), jnp.float32),
            pltpu.SemaphoreType.DMA,
            pltpu.SemaphoreType.DMA,
        ],
    )
    def k(lt_hbm, idx_hbm, a_hbm, b_hbm, out_hbm,
          idx_v, rows_v, cen_v, ob_v, a_v, b_v, sem0, sem1):
        wid = lax.axis_index("s") * NC + lax.axis_index("c")
        base_pt = wid * PW
        bat = base_pt // N
        pltpu.sync_copy(a_hbm.at[bat], a_v)
        pltpu.sync_copy(b_hbm.at[bat], b_v)
        pltpu.sync_copy(idx_hbm.at[pl.ds(wid * NIW, NIW)], idx_v)
        avs = [a_v[0, pl.ds(v * LANES, LANES)] for v in range(NV)]
        bvs = [b_v[0, pl.ds(v * LANES, LANES)] for v in range(NV)]
        z = jnp.zeros((LANES,), jnp.float32)
        sems = (sem0, sem1)

        def issue(ch, s):
            pt0 = base_pt + ch * CP
            pltpu.async_copy(lt_hbm.at[pl.ds(pt0, CP)], cen_v.at[s], sems[s])
            for g in range(NG):
                pltpu.async_copy(lt_hbm.at[idx_v.at[ch * NG + g]],
                                 rows_v.at[s].at[pl.ds(g * 128, 128)], sems[s])

        def drain(s):
            pltpu.make_async_copy(lt_hbm.at[pl.ds(0, CP)], cen_v.at[s],
                                  sems[s]).wait()
            pltpu.make_async_copy(lt_hbm.at[pl.ds(0, RC)], rows_v.at[s],
                                  sems[s]).wait()

        def compute(ch, s):
            def p_body(p, c2):
                evs = [bvs[v] - avs[v] * cen_v[s, p, pl.ds(v * LANES, LANES)]
                       for v in range(NV)]
                accs = [z] * NV
                for j in range(K):
                    r = p * K + j
                    for v in range(NV):
                        t = (avs[v] * rows_v[s, r, pl.ds(v * LANES, LANES)]
                             + evs[v])
                        accs[v] = accs[v] + jnp.maximum(t, 0.0)
                for v in range(NV):
                    ob_v[s * CP + p, pl.ds(v * LANES, LANES)] = accs[v]
                return c2

            lax.fori_loop(0, CP, p_body, 0)

        def flush(ch0):
            pltpu.sync_copy(ob_v, out_hbm.at[pl.ds(base_pt + ch0 * CP, 2 * CP)])

        issue(0, 0)

        def pair_body(g2, c):
            ch0 = g2 * 2
            issue(ch0 + 1, 1)
            drain(0)
            compute(ch0, 0)
            issue(ch0 + 2, 0)
            drain(1)
            compute(ch0 + 1, 1)
            flush(ch0)
            return c

        lax.fori_loop(0, NCH // 2 - 1, pair_body, 0)
        issue(NCH - 1, 1)
        drain(0)
        compute(NCH - 2, 0)
        drain(1)
        compute(NCH - 1, 1)
        flush(NCH - 2)

    return k(localT, idx2d, a8, b8)


def _stage4_call(localT, out_dif, a_cen, b_cen, B, N, COUT):
    NT = 512
    NB = N // NT
    eye = jnp.eye(NT, dtype=jnp.float32)

    def body(lt_ref, od_ref, ac_ref, bc_ref, eye_ref, out_ref):
        cen = jnp.maximum(lt_ref[...] * ac_ref[0] + bc_ref[0], 0.0)
        cenT = lax.dot_general(cen, eye_ref[...], (((0,), (0,)), ((), ())),
                               preferred_element_type=jnp.float32)
        difT = lax.dot_general(od_ref[...], eye_ref[...], (((0,), (0,)), ((), ())),
                               preferred_element_type=jnp.float32)
        out_ref[0, 0:COUT, :] = cenT
        out_ref[0, COUT:2 * COUT, :] = difT

    return pl.pallas_call(
        body,
        grid=(B, NB),
        in_specs=[
            pl.BlockSpec((NT, COUT), lambda b, nb: (b * NB + nb, 0)),
            pl.BlockSpec((NT, COUT), lambda b, nb: (b * NB + nb, 0)),
            pl.BlockSpec((1, 1, COUT), lambda b, nb: (b, 0, 0)),
            pl.BlockSpec((1, 1, COUT), lambda b, nb: (b, 0, 0)),
            pl.BlockSpec((NT, NT), lambda b, nb: (0, 0)),
        ],
        out_specs=pl.BlockSpec((1, 2 * COUT, NT), lambda b, nb: (b, 0, nb)),
        out_shape=jax.ShapeDtypeStruct((B, 2 * COUT, N), jnp.float32),
        compiler_params=pltpu.CompilerParams(
            dimension_semantics=("arbitrary", "arbitrary")),
    )(localT, out_dif, a_cen, b_cen, eye)


def kernel(feature, knn_inds, W1, W2, gamma, beta):
    B, CIN, N = feature.shape
    COUT = W1.shape[0]
    K = knn_inds.shape[2]
    Cg = 2 * COUT // NGROUP

    localT, sum_c, sumsq_c = _stage0_call(feature, W1)

    flat_idx = (knn_inds + (jnp.arange(B, dtype=jnp.int32) * N)[:, None, None])
    idx2d = flat_idx.reshape(B * N * K // 128, 128)

    part = _sc_pass1_call(localT, idx2d, B, N, K, COUT)

    cnt_cen = Cg * N
    cnt_dif = Cg * N * K
    mean_cen = sum_c.reshape(B, 2, Cg).sum(axis=2) / cnt_cen
    var_cen = sumsq_c.reshape(B, 2, Cg).sum(axis=2) / cnt_cen - mean_cen**2
    pw = (part.reshape(NW, 8, 128)[:, 0, :4 * LANES]
          .reshape(B, NW // B, 4, LANES).sum(axis=(1, 3)))
    mean_dif = pw[:, 0:2] / cnt_dif
    var_dif = pw[:, 2:4] / cnt_dif - mean_dif**2
    mean = jnp.concatenate([mean_cen, mean_dif], axis=1)
    inv = 1.0 / jnp.sqrt(jnp.concatenate([var_cen, var_dif], axis=1) + EPS)
    g_of_c = jnp.arange(2 * COUT) // Cg
    a_all = gamma[None, :] * inv[:, g_of_c]
    b_all = beta[None, :] - mean[:, g_of_c] * a_all
    a_cen, a_dif = a_all[:, :COUT], a_all[:, COUT:]
    b_cen, b_dif = b_all[:, :COUT], b_all[:, COUT:]

    aK8 = jnp.broadcast_to((a_dif / K)[:, None, :], (B, 8, COUT))
    bK8 = jnp.broadcast_to((b_dif / K)[:, None, :], (B, 8, COUT))
    out_dif = _sc_pass2_call(localT, idx2d, aK8, bK8, B, N, K, COUT)

    return _stage4_call(localT, out_dif, a_cen.reshape(B, 1, COUT),
                        b_cen.reshape(B, 1, COUT), B, N, COUT)


# speedup vs baseline: 2.4015x; 1.0112x over previous
import functools

import jax
import jax.numpy as jnp
from jax import lax
from jax.experimental import pallas as pl
from jax.experimental.pallas import tpu as pltpu
from jax.experimental.pallas import tpu_sc as plsc

NC = 2
NS = 16
NW = NC * NS
LANES = 16
EPS = 1e-5
NGROUP = 4


def _stage0_call(feature, W1):
    B, CIN, N = feature.shape
    COUT = W1.shape[0]
    NT = 512
    NB = N // NT

    def body(f_ref, w_ref, lt_ref, s_ref, q_ref):
        nb = pl.program_id(1)
        fblk = f_ref[0]
        lt = lax.dot_general(fblk, w_ref[...], (((0,), (1,)), ((), ())),
                             preferred_element_type=jnp.float32)
        lt_ref[...] = lt
        s = jnp.sum(lt, axis=0)[None, None, :]
        q = jnp.sum(lt * lt, axis=0)[None, None, :]

        @pl.when(nb == 0)
        def _():
            s_ref[...] = s
            q_ref[...] = q

        @pl.when(nb != 0)
        def _():
            s_ref[...] = s_ref[...] + s
            q_ref[...] = q_ref[...] + q

    return pl.pallas_call(
        body,
        grid=(B, NB),
        in_specs=[
            pl.BlockSpec((1, CIN, NT), lambda b, nb: (b, 0, nb)),
            pl.BlockSpec((COUT, CIN), lambda b, nb: (0, 0)),
        ],
        out_specs=[
            pl.BlockSpec((NT, COUT), lambda b, nb: (b * NB + nb, 0)),
            pl.BlockSpec((1, 1, COUT), lambda b, nb: (b, 0, 0)),
            pl.BlockSpec((1, 1, COUT), lambda b, nb: (b, 0, 0)),
        ],
        out_shape=[
            jax.ShapeDtypeStruct((B * N, COUT), jnp.float32),
            jax.ShapeDtypeStruct((B, 1, COUT), jnp.float32),
            jax.ShapeDtypeStruct((B, 1, COUT), jnp.float32),
        ],
        compiler_params=pltpu.CompilerParams(
            dimension_semantics=("arbitrary", "arbitrary")),
    )(feature, W1)


def _sc_pass1_call(localT, idx2d, B, N, K, COUT):
    P = B * N
    PW = P // NW
    CP = 16
    RC = CP * K
    NG = RC // 128
    NCH = PW // CP
    NV = COUT // LANES
    mesh = plsc.VectorSubcoreMesh(core_axis_name="c", subcore_axis_name="s",
                                  num_cores=NC, num_subcores=NS)

    NIW = PW * K // 128

    @functools.partial(
        pl.kernel,
        out_type=jax.ShapeDtypeStruct((NW * 8, 128), jnp.float32),
        mesh=mesh,
        scratch_types=[
            pltpu.VMEM((NIW, 128), jnp.int32),
            pltpu.VMEM((2, RC, COUT), jnp.float32),
            pltpu.VMEM((2, CP, COUT), jnp.float32),
            pltpu.VMEM((8, 128), jnp.float32),
            pltpu.VMEM((2 * COUT // LANES, LANES), jnp.float32),
            pltpu.SemaphoreType.DMA,
            pltpu.SemaphoreType.DMA,
        ],
    )
    def k(lt_hbm, idx_hbm, part_hbm, idx_v, rows_v, cen_v, st_v, acc_v,
          sem0, sem1):
        wid = lax.axis_index("s") * NC + lax.axis_index("c")
        base_pt = wid * PW
        pltpu.sync_copy(idx_hbm.at[pl.ds(wid * NIW, NIW)], idx_v)
        sems = (sem0, sem1)
        z = jnp.zeros((LANES,), jnp.float32)
        for v in range(2 * NV):
            acc_v[v, :] = z

        def issue(ch, s):
            pt0 = base_pt + ch * CP
            pltpu.async_copy(lt_hbm.at[pl.ds(pt0, CP)], cen_v.at[s], sems[s])
            for g in range(NG):
                pltpu.async_copy(lt_hbm.at[idx_v.at[ch * NG + g]],
                                 rows_v.at[s].at[pl.ds(g * 128, 128)], sems[s])

        def drain(s):
            pltpu.make_async_copy(lt_hbm.at[pl.ds(0, CP)], cen_v.at[s],
                                  sems[s]).wait()
            pltpu.make_async_copy(lt_hbm.at[pl.ds(0, RC)], rows_v.at[s],
                                  sems[s]).wait()

        def compute(ch, s):
            def p_body(p, c2):
                cvs = [cen_v[s, p, pl.ds(v * LANES, LANES)] for v in range(NV)]
                s1 = [None] * NV
                s2 = [None] * NV
                for j in range(K):
                    r = p * K + j
                    for v in range(NV):
                        dv = rows_v[s, r, pl.ds(v * LANES, LANES)] - cvs[v]
                        sq = dv * dv
                        s1[v] = dv if j == 0 else s1[v] + dv
                        s2[v] = sq if j == 0 else s2[v] + sq
                for v in range(NV):
                    plsc.addupdate(acc_v.at[v], s1[v])
                    plsc.addupdate(acc_v.at[NV + v], s2[v])
                return c2

            lax.fori_loop(0, CP, p_body, 0)

        issue(0, 0)

        def pair_body(g2, c):
            ch0 = g2 * 2
            issue(ch0 + 1, 1)
            drain(0)
            compute(ch0, 0)
            issue(ch0 + 2, 0)
            drain(1)
            compute(ch0 + 1, 1)
            return c

        lax.fori_loop(0, NCH // 2 - 1, pair_body, 0)
        issue(NCH - 1, 1)
        drain(0)
        compute(NCH - 2, 0)
        drain(1)
        compute(NCH - 1, 1)

        h = NV // 2
        carry = [acc_v[v, :] for v in range(2 * NV)]
        s1a = sum(carry[1:h], carry[0])
        s1b = sum(carry[h + 1:NV], carry[h])
        s2a = sum(carry[NV + 1:NV + h], carry[NV])
        s2b = sum(carry[NV + h + 1:], carry[NV + h])
        st_v[0, pl.ds(0, LANES)] = s1a
        st_v[0, pl.ds(LANES, LANES)] = s1b
        st_v[0, pl.ds(2 * LANES, LANES)] = s2a
        st_v[0, pl.ds(3 * LANES, LANES)] = s2b
        pltpu.sync_copy(st_v, part_hbm.at[pl.ds(wid * 8, 8)])

    return k(localT, idx2d)


def _sc_pass2_call(localT, idx2d, a8, b8, B, N, K, COUT):
    P = B * N
    PW = P // NW
    CP = 16
    RC = CP * K
    NG = RC // 128
    NCH = PW // CP
    NV = COUT // LANES
    mesh = plsc.VectorSubcoreMesh(core_axis_name="c", subcore_axis_name="s",
                                  num_cores=NC, num_subcores=NS)

    NIW = PW * K // 128

    @functools.partial(
        pl.kernel,
        out_type=jax.ShapeDtypeStruct((P, COUT), jnp.float32),
        mesh=mesh,
        scratch_types=[
            pltpu.VMEM((NIW, 128), jnp.int32),
            pltpu.VMEM((2, RC, COUT), jnp.float32),
            pltpu.VMEM((2, CP, COUT), jnp.float32),
            pltpu.VMEM((2 * CP, COUT), jnp.float32),
            pltpu.VMEM((8, COUT), jnp.float32),
            pltpu.VMEM((8, COUT), jnp.float32),
            pltpu.SemaphoreType.DMA,
            pltpu.SemaphoreType.DMA,
        ],
    )
    def k(lt_hbm, idx_hbm, a_hbm, b_hbm, out_hbm,
          idx_v, rows_v, cen_v, ob_v, a_v, b_v, sem0, sem1):
        wid = lax.axis_index("s") * NC + lax.axis_index("c")
        base_pt = wid * PW
        bat = base_pt // N
        pltpu.sync_copy(a_hbm.at[bat], a_v)
        pltpu.sync_copy(b_hbm.at[bat], b_v)
        pltpu.sync_copy(idx_hbm.at[pl.ds(wid * NIW, NIW)], idx_v)
        avs = [a_v[0, pl.ds(v * LANES, LANES)] for v in range(NV)]
        bvs = [b_v[0, pl.ds(v * LANES, LANES)] for v in range(NV)]
        z = jnp.zeros((LANES,), jnp.float32)
        sems = (sem0, sem1)

        def issue(ch, s):
            pt0 = base_pt + ch * CP
            pltpu.async_copy(lt_hbm.at[pl.ds(pt0, CP)], cen_v.at[s], sems[s])
            for g in range(NG):
                pltpu.async_copy(lt_hbm.at[idx_v.at[ch * NG + g]],
                                 rows_v.at[s].at[pl.ds(g * 128, 128)], sems[s])

        def drain(s):
            pltpu.make_async_copy(lt_hbm.at[pl.ds(0, CP)], cen_v.at[s],
                                  sems[s]).wait()
            pltpu.make_async_copy(lt_hbm.at[pl.ds(0, RC)], rows_v.at[s],
                                  sems[s]).wait()

        def compute(ch, s):
            def p_body(p, c2):
                evs = [bvs[v] - avs[v] * cen_v[s, p, pl.ds(v * LANES, LANES)]
                       for v in range(NV)]
                accs = [z] * NV
                for j in range(K):
                    r = p * K + j
                    for v in range(NV):
                        t = (avs[v] * rows_v[s, r, pl.ds(v * LANES, LANES)]
                             + evs[v])
                        accs[v] = accs[v] + jnp.maximum(t, 0.0)
                for v in range(NV):
                    ob_v[s * CP + p, pl.ds(v * LANES, LANES)] = accs[v]
                return c2

            lax.fori_loop(0, CP, p_body, 0)

        def flush(ch0):
            pltpu.sync_copy(ob_v, out_hbm.at[pl.ds(base_pt + ch0 * CP, 2 * CP)])

        issue(0, 0)

        def pair_body(g2, c):
            ch0 = g2 * 2
            issue(ch0 + 1, 1)
            drain(0)
            compute(ch0, 0)
            issue(ch0 + 2, 0)
            drain(1)
            compute(ch0 + 1, 1)
            flush(ch0)
            return c

        lax.fori_loop(0, NCH // 2 - 1, pair_body, 0)
        issue(NCH - 1, 1)
        drain(0)
        compute(NCH - 2, 0)
        drain(1)
        compute(NCH - 1, 1)
        flush(NCH - 2)

    return k(localT, idx2d, a8, b8)


def _stage4_call(localT, out_dif, a_cen, b_cen, B, N, COUT):
    NT = 512
    NB = N // NT

    def body(lt_ref, od_ref, ac_ref, bc_ref, out_ref):
        cen = jnp.maximum(lt_ref[...] * ac_ref[0] + bc_ref[0], 0.0)
        out_ref[0, 0:COUT, :] = cen.T
        out_ref[0, COUT:2 * COUT, :] = od_ref[...].T

    return pl.pallas_call(
        body,
        grid=(B, NB),
        in_specs=[
            pl.BlockSpec((NT, COUT), lambda b, nb: (b * NB + nb, 0)),
            pl.BlockSpec((NT, COUT), lambda b, nb: (b * NB + nb, 0)),
            pl.BlockSpec((1, 1, COUT), lambda b, nb: (b, 0, 0)),
            pl.BlockSpec((1, 1, COUT), lambda b, nb: (b, 0, 0)),
        ],
        out_specs=pl.BlockSpec((1, 2 * COUT, NT), lambda b, nb: (b, 0, nb)),
        out_shape=jax.ShapeDtypeStruct((B, 2 * COUT, N), jnp.float32),
        compiler_params=pltpu.CompilerParams(
            dimension_semantics=("arbitrary", "arbitrary")),
    )(localT, out_dif, a_cen, b_cen)


def kernel(feature, knn_inds, W1, W2, gamma, beta):
    B, CIN, N = feature.shape
    COUT = W1.shape[0]
    K = knn_inds.shape[2]
    Cg = 2 * COUT // NGROUP

    localT, sum_c, sumsq_c = _stage0_call(feature, W1)

    flat_idx = (knn_inds + (jnp.arange(B, dtype=jnp.int32) * N)[:, None, None])
    idx2d = flat_idx.reshape(B * N * K // 128, 128)

    part = _sc_pass1_call(localT, idx2d, B, N, K, COUT)

    cnt_cen = Cg * N
    cnt_dif = Cg * N * K
    mean_cen = sum_c.reshape(B, 2, Cg).sum(axis=2) / cnt_cen
    var_cen = sumsq_c.reshape(B, 2, Cg).sum(axis=2) / cnt_cen - mean_cen**2
    pw = (part.reshape(NW, 8, 128)[:, 0, :4 * LANES]
          .reshape(B, NW // B, 4, LANES).sum(axis=(1, 3)))
    mean_dif = pw[:, 0:2] / cnt_dif
    var_dif = pw[:, 2:4] / cnt_dif - mean_dif**2
    mean = jnp.concatenate([mean_cen, mean_dif], axis=1)
    inv = 1.0 / jnp.sqrt(jnp.concatenate([var_cen, var_dif], axis=1) + EPS)
    g_of_c = jnp.arange(2 * COUT) // Cg
    a_all = gamma[None, :] * inv[:, g_of_c]
    b_all = beta[None, :] - mean[:, g_of_c] * a_all
    a_cen, a_dif = a_all[:, :COUT], a_all[:, COUT:]
    b_cen, b_dif = b_all[:, :COUT], b_all[:, COUT:]

    aK8 = jnp.broadcast_to((a_dif / K)[:, None, :], (B, 8, COUT))
    bK8 = jnp.broadcast_to((b_dif / K)[:, None, :], (B, 8, COUT))
    out_dif = _sc_pass2_call(localT, idx2d, aK8, bK8, B, N, K, COUT)

    return _stage4_call(localT, out_dif, a_cen.reshape(B, 1, COUT),
                        b_cen.reshape(B, 1, COUT), B, N, COUT)


# speedup vs baseline: 2.5809x; 1.0747x over previous
import functools

import jax
import jax.numpy as jnp
from jax import lax
from jax.experimental import pallas as pl
from jax.experimental.pallas import tpu as pltpu
from jax.experimental.pallas import tpu_sc as plsc

NC = 2
NS = 16
NW = NC * NS
LANES = 16
EPS = 1e-5
NGROUP = 4


def _stage0_call(feature, W1):
    B, CIN, N = feature.shape
    COUT = W1.shape[0]
    NT = 1024
    NB = N // NT

    def body(f_ref, w_ref, lt_ref, s_ref, q_ref):
        nb = pl.program_id(1)
        fblk = f_ref[0]
        lt = lax.dot_general(fblk, w_ref[...], (((0,), (1,)), ((), ())),
                             preferred_element_type=jnp.float32)
        lt_ref[...] = lt
        s = jnp.sum(lt, axis=0)[None, None, :]
        q = jnp.sum(lt * lt, axis=0)[None, None, :]

        @pl.when(nb == 0)
        def _():
            s_ref[...] = s
            q_ref[...] = q

        @pl.when(nb != 0)
        def _():
            s_ref[...] = s_ref[...] + s
            q_ref[...] = q_ref[...] + q

    return pl.pallas_call(
        body,
        grid=(B, NB),
        in_specs=[
            pl.BlockSpec((1, CIN, NT), lambda b, nb: (b, 0, nb)),
            pl.BlockSpec((COUT, CIN), lambda b, nb: (0, 0)),
        ],
        out_specs=[
            pl.BlockSpec((NT, COUT), lambda b, nb: (b * NB + nb, 0)),
            pl.BlockSpec((1, 1, COUT), lambda b, nb: (b, 0, 0)),
            pl.BlockSpec((1, 1, COUT), lambda b, nb: (b, 0, 0)),
        ],
        out_shape=[
            jax.ShapeDtypeStruct((B * N, COUT), jnp.float32),
            jax.ShapeDtypeStruct((B, 1, COUT), jnp.float32),
            jax.ShapeDtypeStruct((B, 1, COUT), jnp.float32),
        ],
        compiler_params=pltpu.CompilerParams(
            dimension_semantics=("arbitrary", "arbitrary")),
    )(feature, W1)


def _sc_pass1_call(localT, idx2d, B, N, K, COUT):
    P = B * N
    PW = P // NW
    CP = 16
    RC = CP * K
    NG = RC // 128
    NCH = PW // CP
    NV = COUT // LANES
    mesh = plsc.VectorSubcoreMesh(core_axis_name="c", subcore_axis_name="s",
                                  num_cores=NC, num_subcores=NS)

    NIW = PW * K // 128

    @functools.partial(
        pl.kernel,
        out_type=jax.ShapeDtypeStruct((NW * 8, 128), jnp.float32),
        mesh=mesh,
        scratch_types=[
            pltpu.VMEM((NIW, 128), jnp.int32),
            pltpu.VMEM((2, RC, COUT), jnp.float32),
            pltpu.VMEM((2, CP, COUT), jnp.float32),
            pltpu.VMEM((8, 128), jnp.float32),
            pltpu.VMEM((2 * COUT // LANES, LANES), jnp.float32),
            pltpu.SemaphoreType.DMA,
            pltpu.SemaphoreType.DMA,
        ],
    )
    def k(lt_hbm, idx_hbm, part_hbm, idx_v, rows_v, cen_v, st_v, acc_v,
          sem0, sem1):
        wid = lax.axis_index("s") * NC + lax.axis_index("c")
        base_pt = wid * PW
        pltpu.sync_copy(idx_hbm.at[pl.ds(wid * NIW, NIW)], idx_v)
        sems = (sem0, sem1)
        z = jnp.zeros((LANES,), jnp.float32)
        for v in range(2 * NV):
            acc_v[v, :] = z

        def issue(ch, s):
            pt0 = base_pt + ch * CP
            pltpu.async_copy(lt_hbm.at[pl.ds(pt0, CP)], cen_v.at[s], sems[s])
            for g in range(NG):
                pltpu.async_copy(lt_hbm.at[idx_v.at[ch * NG + g]],
                                 rows_v.at[s].at[pl.ds(g * 128, 128)], sems[s])

        def drain(s):
            pltpu.make_async_copy(lt_hbm.at[pl.ds(0, CP)], cen_v.at[s],
                                  sems[s]).wait()
            pltpu.make_async_copy(lt_hbm.at[pl.ds(0, RC)], rows_v.at[s],
                                  sems[s]).wait()

        def compute(ch, s):
            def p_body(p, c2):
                cvs = [cen_v[s, p, pl.ds(v * LANES, LANES)] for v in range(NV)]
                s1 = [None] * NV
                s2 = [None] * NV
                for j in range(K):
                    r = p * K + j
                    for v in range(NV):
                        dv = rows_v[s, r, pl.ds(v * LANES, LANES)] - cvs[v]
                        sq = dv * dv
                        s1[v] = dv if j == 0 else s1[v] + dv
                        s2[v] = sq if j == 0 else s2[v] + sq
                for v in range(NV):
                    plsc.addupdate(acc_v.at[v], s1[v])
                    plsc.addupdate(acc_v.at[NV + v], s2[v])
                return c2

            lax.fori_loop(0, CP, p_body, 0)

        issue(0, 0)

        def pair_body(g2, c):
            ch0 = g2 * 2
            issue(ch0 + 1, 1)
            drain(0)
            compute(ch0, 0)
            issue(ch0 + 2, 0)
            drain(1)
            compute(ch0 + 1, 1)
            return c

        lax.fori_loop(0, NCH // 2 - 1, pair_body, 0)
        issue(NCH - 1, 1)
        drain(0)
        compute(NCH - 2, 0)
        drain(1)
        compute(NCH - 1, 1)

        h = NV // 2
        carry = [acc_v[v, :] for v in range(2 * NV)]
        s1a = sum(carry[1:h], carry[0])
        s1b = sum(carry[h + 1:NV], carry[h])
        s2a = sum(carry[NV + 1:NV + h], carry[NV])
        s2b = sum(carry[NV + h + 1:], carry[NV + h])
        st_v[0, pl.ds(0, LANES)] = s1a
        st_v[0, pl.ds(LANES, LANES)] = s1b
        st_v[0, pl.ds(2 * LANES, LANES)] = s2a
        st_v[0, pl.ds(3 * LANES, LANES)] = s2b
        pltpu.sync_copy(st_v, part_hbm.at[pl.ds(wid * 8, 8)])

    return k(localT, idx2d)


def _sc_pass2_call(localT, idx2d, a8, b8, B, N, K, COUT):
    P = B * N
    PW = P // NW
    CP = 16
    RC = CP * K
    NG = RC // 128
    NCH = PW // CP
    NV = COUT // LANES
    mesh = plsc.VectorSubcoreMesh(core_axis_name="c", subcore_axis_name="s",
                                  num_cores=NC, num_subcores=NS)

    NIW = PW * K // 128

    @functools.partial(
        pl.kernel,
        out_type=jax.ShapeDtypeStruct((P, COUT), jnp.float32),
        mesh=mesh,
        scratch_types=[
            pltpu.VMEM((NIW, 128), jnp.int32),
            pltpu.VMEM((2, RC, COUT), jnp.float32),
            pltpu.VMEM((2, CP, COUT), jnp.float32),
            pltpu.VMEM((2 * CP, COUT), jnp.float32),
            pltpu.VMEM((8, COUT), jnp.float32),
            pltpu.VMEM((8, COUT), jnp.float32),
            pltpu.SemaphoreType.DMA,
            pltpu.SemaphoreType.DMA,
        ],
    )
    def k(lt_hbm, idx_hbm, a_hbm, b_hbm, out_hbm,
          idx_v, rows_v, cen_v, ob_v, a_v, b_v, sem0, sem1):
        wid = lax.axis_index("s") * NC + lax.axis_index("c")
        base_pt = wid * PW
        bat = base_pt // N
        pltpu.sync_copy(a_hbm.at[bat], a_v)
        pltpu.sync_copy(b_hbm.at[bat], b_v)
        pltpu.sync_copy(idx_hbm.at[pl.ds(wid * NIW, NIW)], idx_v)
        avs = [a_v[0, pl.ds(v * LANES, LANES)] for v in range(NV)]
        bvs = [b_v[0, pl.ds(v * LANES, LANES)] for v in range(NV)]
        z = jnp.zeros((LANES,), jnp.float32)
        sems = (sem0, sem1)

        def issue(ch, s):
            pt0 = base_pt + ch * CP
            pltpu.async_copy(lt_hbm.at[pl.ds(pt0, CP)], cen_v.at[s], sems[s])
            for g in range(NG):
                pltpu.async_copy(lt_hbm.at[idx_v.at[ch * NG + g]],
                                 rows_v.at[s].at[pl.ds(g * 128, 128)], sems[s])

        def drain(s):
            pltpu.make_async_copy(lt_hbm.at[pl.ds(0, CP)], cen_v.at[s],
                                  sems[s]).wait()
            pltpu.make_async_copy(lt_hbm.at[pl.ds(0, RC)], rows_v.at[s],
                                  sems[s]).wait()

        def compute(ch, s):
            def p_body(p, c2):
                evs = [bvs[v] - avs[v] * cen_v[s, p, pl.ds(v * LANES, LANES)]
                       for v in range(NV)]
                accs = [z] * NV
                for j in range(K):
                    r = p * K + j
                    for v in range(NV):
                        t = (avs[v] * rows_v[s, r, pl.ds(v * LANES, LANES)]
                             + evs[v])
                        accs[v] = accs[v] + jnp.maximum(t, 0.0)
                for v in range(NV):
                    ob_v[s * CP + p, pl.ds(v * LANES, LANES)] = accs[v]
                return c2

            lax.fori_loop(0, CP, p_body, 0)

        def flush(ch0):
            pltpu.sync_copy(ob_v, out_hbm.at[pl.ds(base_pt + ch0 * CP, 2 * CP)])

        issue(0, 0)

        def pair_body(g2, c):
            ch0 = g2 * 2
            issue(ch0 + 1, 1)
            drain(0)
            compute(ch0, 0)
            issue(ch0 + 2, 0)
            drain(1)
            compute(ch0 + 1, 1)
            flush(ch0)
            return c

        lax.fori_loop(0, NCH // 2 - 1, pair_body, 0)
        issue(NCH - 1, 1)
        drain(0)
        compute(NCH - 2, 0)
        drain(1)
        compute(NCH - 1, 1)
        flush(NCH - 2)

    return k(localT, idx2d, a8, b8)


def _stage4_call(localT, out_dif, a_cen, b_cen, B, N, COUT):
    NT = 1024
    NB = N // NT

    def body(lt_ref, od_ref, ac_ref, bc_ref, out_ref):
        cen = jnp.maximum(lt_ref[...] * ac_ref[0] + bc_ref[0], 0.0)
        out_ref[0, 0:COUT, :] = cen.T
        out_ref[0, COUT:2 * COUT, :] = od_ref[...].T

    return pl.pallas_call(
        body,
        grid=(B, NB),
        in_specs=[
            pl.BlockSpec((NT, COUT), lambda b, nb: (b * NB + nb, 0)),
            pl.BlockSpec((NT, COUT), lambda b, nb: (b * NB + nb, 0)),
            pl.BlockSpec((1, 1, COUT), lambda b, nb: (b, 0, 0)),
            pl.BlockSpec((1, 1, COUT), lambda b, nb: (b, 0, 0)),
        ],
        out_specs=pl.BlockSpec((1, 2 * COUT, NT), lambda b, nb: (b, 0, nb)),
        out_shape=jax.ShapeDtypeStruct((B, 2 * COUT, N), jnp.float32),
        compiler_params=pltpu.CompilerParams(
            dimension_semantics=("arbitrary", "arbitrary")),
    )(localT, out_dif, a_cen, b_cen)


def kernel(feature, knn_inds, W1, W2, gamma, beta):
    B, CIN, N = feature.shape
    COUT = W1.shape[0]
    K = knn_inds.shape[2]
    Cg = 2 * COUT // NGROUP

    localT, sum_c, sumsq_c = _stage0_call(feature, W1)

    flat_idx = (knn_inds + (jnp.arange(B, dtype=jnp.int32) * N)[:, None, None])
    idx2d = flat_idx.reshape(B * N * K // 128, 128)

    part = _sc_pass1_call(localT, idx2d, B, N, K, COUT)

    cnt_cen = Cg * N
    cnt_dif = Cg * N * K
    mean_cen = sum_c.reshape(B, 2, Cg).sum(axis=2) / cnt_cen
    var_cen = sumsq_c.reshape(B, 2, Cg).sum(axis=2) / cnt_cen - mean_cen**2
    pw = (part.reshape(NW, 8, 128)[:, 0, :4 * LANES]
          .reshape(B, NW // B, 4, LANES).sum(axis=(1, 3)))
    mean_dif = pw[:, 0:2] / cnt_dif
    var_dif = pw[:, 2:4] / cnt_dif - mean_dif**2
    mean = jnp.concatenate([mean_cen, mean_dif], axis=1)
    inv = 1.0 / jnp.sqrt(jnp.concatenate([var_cen, var_dif], axis=1) + EPS)
    g_of_c = jnp.arange(2 * COUT) // Cg
    a_all = gamma[None, :] * inv[:, g_of_c]
    b_all = beta[None, :] - mean[:, g_of_c] * a_all
    a_cen, a_dif = a_all[:, :COUT], a_all[:, COUT:]
    b_cen, b_dif = b_all[:, :COUT], b_all[:, COUT:]

    aK8 = jnp.broadcast_to((a_dif / K)[:, None, :], (B, 8, COUT))
    bK8 = jnp.broadcast_to((b_dif / K)[:, None, :], (B, 8, COUT))
    out_dif = _sc_pass2_call(localT, idx2d, aK8, bK8, B, N, K, COUT)

    return _stage4_call(localT, out_dif, a_cen.reshape(B, 1, COUT),
                        b_cen.reshape(B, 1, COUT), B, N, COUT)


# speedup vs baseline: 2.6862x; 1.0408x over previous
import functools

import jax
import jax.numpy as jnp
from jax import lax
from jax.experimental import pallas as pl
from jax.experimental.pallas import tpu as pltpu
from jax.experimental.pallas import tpu_sc as plsc

NC = 2
NS = 16
NW = NC * NS
LANES = 16
EPS = 1e-5
NGROUP = 4


def _stage0_call(feature, W1):
    B, CIN, N = feature.shape
    COUT = W1.shape[0]
    NT = 2048
    NB = N // NT

    def body(f_ref, w_ref, lt_ref, s_ref, q_ref):
        nb = pl.program_id(1)
        fblk = f_ref[0]
        lt = lax.dot_general(fblk, w_ref[...], (((0,), (1,)), ((), ())),
                             preferred_element_type=jnp.float32)
        lt_ref[...] = lt
        s = jnp.sum(lt, axis=0)[None, None, :]
        q = jnp.sum(lt * lt, axis=0)[None, None, :]

        @pl.when(nb == 0)
        def _():
            s_ref[...] = s
            q_ref[...] = q

        @pl.when(nb != 0)
        def _():
            s_ref[...] = s_ref[...] + s
            q_ref[...] = q_ref[...] + q

    return pl.pallas_call(
        body,
        grid=(B, NB),
        in_specs=[
            pl.BlockSpec((1, CIN, NT), lambda b, nb: (b, 0, nb)),
            pl.BlockSpec((COUT, CIN), lambda b, nb: (0, 0)),
        ],
        out_specs=[
            pl.BlockSpec((NT, COUT), lambda b, nb: (b * NB + nb, 0)),
            pl.BlockSpec((1, 1, COUT), lambda b, nb: (b, 0, 0)),
            pl.BlockSpec((1, 1, COUT), lambda b, nb: (b, 0, 0)),
        ],
        out_shape=[
            jax.ShapeDtypeStruct((B * N, COUT), jnp.float32),
            jax.ShapeDtypeStruct((B, 1, COUT), jnp.float32),
            jax.ShapeDtypeStruct((B, 1, COUT), jnp.float32),
        ],
        compiler_params=pltpu.CompilerParams(
            dimension_semantics=("arbitrary", "arbitrary")),
    )(feature, W1)


def _sc_pass1_call(localT, idx2d, B, N, K, COUT):
    P = B * N
    PW = P // NW
    CP = 16
    RC = CP * K
    NG = RC // 128
    NCH = PW // CP
    NV = COUT // LANES
    mesh = plsc.VectorSubcoreMesh(core_axis_name="c", subcore_axis_name="s",
                                  num_cores=NC, num_subcores=NS)

    NIW = PW * K // 128

    @functools.partial(
        pl.kernel,
        out_type=jax.ShapeDtypeStruct((NW * 8, 128), jnp.float32),
        mesh=mesh,
        scratch_types=[
            pltpu.VMEM((NIW, 128), jnp.int32),
            pltpu.VMEM((2, RC, COUT), jnp.float32),
            pltpu.VMEM((2, CP, COUT), jnp.float32),
            pltpu.VMEM((8, 128), jnp.float32),
            pltpu.VMEM((2 * COUT // LANES, LANES), jnp.float32),
            pltpu.SemaphoreType.DMA,
            pltpu.SemaphoreType.DMA,
        ],
    )
    def k(lt_hbm, idx_hbm, part_hbm, idx_v, rows_v, cen_v, st_v, acc_v,
          sem0, sem1):
        wid = lax.axis_index("s") * NC + lax.axis_index("c")
        base_pt = wid * PW
        pltpu.sync_copy(idx_hbm.at[pl.ds(wid * NIW, NIW)], idx_v)
        sems = (sem0, sem1)
        z = jnp.zeros((LANES,), jnp.float32)
        for v in range(2 * NV):
            acc_v[v, :] = z

        def issue(ch, s):
            pt0 = base_pt + ch * CP
            pltpu.async_copy(lt_hbm.at[pl.ds(pt0, CP)], cen_v.at[s], sems[s])
            for g in range(NG):
                pltpu.async_copy(lt_hbm.at[idx_v.at[ch * NG + g]],
                                 rows_v.at[s].at[pl.ds(g * 128, 128)], sems[s])

        def drain(s):
            pltpu.make_async_copy(lt_hbm.at[pl.ds(0, CP)], cen_v.at[s],
                                  sems[s]).wait()
            pltpu.make_async_copy(lt_hbm.at[pl.ds(0, RC)], rows_v.at[s],
                                  sems[s]).wait()

        def compute(ch, s):
            def p_body(p, c2):
                cvs = [cen_v[s, p, pl.ds(v * LANES, LANES)] for v in range(NV)]
                s1 = [None] * NV
                s2 = [None] * NV
                for j in range(K):
                    r = p * K + j
                    for v in range(NV):
                        dv = rows_v[s, r, pl.ds(v * LANES, LANES)] - cvs[v]
                        sq = dv * dv
                        s1[v] = dv if j == 0 else s1[v] + dv
                        s2[v] = sq if j == 0 else s2[v] + sq
                for v in range(NV):
                    plsc.addupdate(acc_v.at[v], s1[v])
                    plsc.addupdate(acc_v.at[NV + v], s2[v])
                return c2

            lax.fori_loop(0, CP, p_body, 0)

        issue(0, 0)

        def pair_body(g2, c):
            ch0 = g2 * 2
            issue(ch0 + 1, 1)
            drain(0)
            compute(ch0, 0)
            issue(ch0 + 2, 0)
            drain(1)
            compute(ch0 + 1, 1)
            return c

        lax.fori_loop(0, NCH // 2 - 1, pair_body, 0)
        issue(NCH - 1, 1)
        drain(0)
        compute(NCH - 2, 0)
        drain(1)
        compute(NCH - 1, 1)

        h = NV // 2
        carry = [acc_v[v, :] for v in range(2 * NV)]
        s1a = sum(carry[1:h], carry[0])
        s1b = sum(carry[h + 1:NV], carry[h])
        s2a = sum(carry[NV + 1:NV + h], carry[NV])
        s2b = sum(carry[NV + h + 1:], carry[NV + h])
        st_v[0, pl.ds(0, LANES)] = s1a
        st_v[0, pl.ds(LANES, LANES)] = s1b
        st_v[0, pl.ds(2 * LANES, LANES)] = s2a
        st_v[0, pl.ds(3 * LANES, LANES)] = s2b
        pltpu.sync_copy(st_v, part_hbm.at[pl.ds(wid * 8, 8)])

    return k(localT, idx2d)


def _sc_pass2_call(localT, idx2d, a8, b8, B, N, K, COUT):
    P = B * N
    PW = P // NW
    CP = 16
    RC = CP * K
    NG = RC // 128
    NCH = PW // CP
    NV = COUT // LANES
    mesh = plsc.VectorSubcoreMesh(core_axis_name="c", subcore_axis_name="s",
                                  num_cores=NC, num_subcores=NS)

    NIW = PW * K // 128

    @functools.partial(
        pl.kernel,
        out_type=jax.ShapeDtypeStruct((P, COUT), jnp.float32),
        mesh=mesh,
        scratch_types=[
            pltpu.VMEM((NIW, 128), jnp.int32),
            pltpu.VMEM((2, RC, COUT), jnp.float32),
            pltpu.VMEM((2, CP, COUT), jnp.float32),
            pltpu.VMEM((2 * CP, COUT), jnp.float32),
            pltpu.VMEM((8, COUT), jnp.float32),
            pltpu.VMEM((8, COUT), jnp.float32),
            pltpu.SemaphoreType.DMA,
            pltpu.SemaphoreType.DMA,
        ],
    )
    def k(lt_hbm, idx_hbm, a_hbm, b_hbm, out_hbm,
          idx_v, rows_v, cen_v, ob_v, a_v, b_v, sem0, sem1):
        wid = lax.axis_index("s") * NC + lax.axis_index("c")
        base_pt = wid * PW
        bat = base_pt // N
        pltpu.sync_copy(a_hbm.at[bat], a_v)
        pltpu.sync_copy(b_hbm.at[bat], b_v)
        pltpu.sync_copy(idx_hbm.at[pl.ds(wid * NIW, NIW)], idx_v)
        avs = [a_v[0, pl.ds(v * LANES, LANES)] for v in range(NV)]
        bvs = [b_v[0, pl.ds(v * LANES, LANES)] for v in range(NV)]
        z = jnp.zeros((LANES,), jnp.float32)
        sems = (sem0, sem1)

        def issue(ch, s):
            pt0 = base_pt + ch * CP
            pltpu.async_copy(lt_hbm.at[pl.ds(pt0, CP)], cen_v.at[s], sems[s])
            for g in range(NG):
                pltpu.async_copy(lt_hbm.at[idx_v.at[ch * NG + g]],
                                 rows_v.at[s].at[pl.ds(g * 128, 128)], sems[s])

        def drain(s):
            pltpu.make_async_copy(lt_hbm.at[pl.ds(0, CP)], cen_v.at[s],
                                  sems[s]).wait()
            pltpu.make_async_copy(lt_hbm.at[pl.ds(0, RC)], rows_v.at[s],
                                  sems[s]).wait()

        def compute(ch, s):
            def p_body(p, c2):
                evs = [bvs[v] - avs[v] * cen_v[s, p, pl.ds(v * LANES, LANES)]
                       for v in range(NV)]
                accs = [z] * NV
                for j in range(K):
                    r = p * K + j
                    for v in range(NV):
                        t = (avs[v] * rows_v[s, r, pl.ds(v * LANES, LANES)]
                             + evs[v])
                        accs[v] = accs[v] + jnp.maximum(t, 0.0)
                for v in range(NV):
                    ob_v[s * CP + p, pl.ds(v * LANES, LANES)] = accs[v]
                return c2

            lax.fori_loop(0, CP, p_body, 0)

        def flush(ch0):
            pltpu.sync_copy(ob_v, out_hbm.at[pl.ds(base_pt + ch0 * CP, 2 * CP)])

        issue(0, 0)

        def pair_body(g2, c):
            ch0 = g2 * 2
            issue(ch0 + 1, 1)
            drain(0)
            compute(ch0, 0)
            issue(ch0 + 2, 0)
            drain(1)
            compute(ch0 + 1, 1)
            flush(ch0)
            return c

        lax.fori_loop(0, NCH // 2 - 1, pair_body, 0)
        issue(NCH - 1, 1)
        drain(0)
        compute(NCH - 2, 0)
        drain(1)
        compute(NCH - 1, 1)
        flush(NCH - 2)

    return k(localT, idx2d, a8, b8)


def _stage4_call(localT, out_dif, a_cen, b_cen, B, N, COUT):
    NT = 2048
    NB = N // NT

    def body(lt_ref, od_ref, ac_ref, bc_ref, out_ref):
        cen = jnp.maximum(lt_ref[...] * ac_ref[0] + bc_ref[0], 0.0)
        out_ref[0, 0:COUT, :] = cen.T
        out_ref[0, COUT:2 * COUT, :] = od_ref[...].T

    return pl.pallas_call(
        body,
        grid=(B, NB),
        in_specs=[
            pl.BlockSpec((NT, COUT), lambda b, nb: (b * NB + nb, 0)),
            pl.BlockSpec((NT, COUT), lambda b, nb: (b * NB + nb, 0)),
            pl.BlockSpec((1, 1, COUT), lambda b, nb: (b, 0, 0)),
            pl.BlockSpec((1, 1, COUT), lambda b, nb: (b, 0, 0)),
        ],
        out_specs=pl.BlockSpec((1, 2 * COUT, NT), lambda b, nb: (b, 0, nb)),
        out_shape=jax.ShapeDtypeStruct((B, 2 * COUT, N), jnp.float32),
        compiler_params=pltpu.CompilerParams(
            dimension_semantics=("arbitrary", "arbitrary")),
    )(localT, out_dif, a_cen, b_cen)


def kernel(feature, knn_inds, W1, W2, gamma, beta):
    B, CIN, N = feature.shape
    COUT = W1.shape[0]
    K = knn_inds.shape[2]
    Cg = 2 * COUT // NGROUP

    localT, sum_c, sumsq_c = _stage0_call(feature, W1)

    flat_idx = (knn_inds + (jnp.arange(B, dtype=jnp.int32) * N)[:, None, None])
    idx2d = flat_idx.reshape(B * N * K // 128, 128)

    part = _sc_pass1_call(localT, idx2d, B, N, K, COUT)

    cnt_cen = Cg * N
    cnt_dif = Cg * N * K
    mean_cen = sum_c.reshape(B, 2, Cg).sum(axis=2) / cnt_cen
    var_cen = sumsq_c.reshape(B, 2, Cg).sum(axis=2) / cnt_cen - mean_cen**2
    pw = (part.reshape(NW, 8, 128)[:, 0, :4 * LANES]
          .reshape(B, NW // B, 4, LANES).sum(axis=(1, 3)))
    mean_dif = pw[:, 0:2] / cnt_dif
    var_dif = pw[:, 2:4] / cnt_dif - mean_dif**2
    mean = jnp.concatenate([mean_cen, mean_dif], axis=1)
    inv = 1.0 / jnp.sqrt(jnp.concatenate([var_cen, var_dif], axis=1) + EPS)
    g_of_c = jnp.arange(2 * COUT) // Cg
    a_all = gamma[None, :] * inv[:, g_of_c]
    b_all = beta[None, :] - mean[:, g_of_c] * a_all
    a_cen, a_dif = a_all[:, :COUT], a_all[:, COUT:]
    b_cen, b_dif = b_all[:, :COUT], b_all[:, COUT:]

    aK8 = jnp.broadcast_to((a_dif / K)[:, None, :], (B, 8, COUT))
    bK8 = jnp.broadcast_to((b_dif / K)[:, None, :], (B, 8, COUT))
    out_dif = _sc_pass2_call(localT, idx2d, aK8, bK8, B, N, K, COUT)

    return _stage4_call(localT, out_dif, a_cen.reshape(B, 1, COUT),
                        b_cen.reshape(B, 1, COUT), B, N, COUT)
